# baseline (speedup 1.0000x reference)
import functools

import jax
import jax.numpy as jnp
from jax import lax
from jax.experimental import pallas as pl
from jax.experimental.pallas import tpu as pltpu
from jax.experimental.pallas import tpu_sc as plsc

EMBED_DIM = 32
E_BLK = EMBED_DIM // 8
NUM_CORES = 2
NUM_SUBCORES = 16
NUM_WORKERS = NUM_CORES * NUM_SUBCORES
B_BLK = 128
T_CHUNK = 10
LANES = 16


@functools.lru_cache(maxsize=None)
def _make_gather(n_b, n_t):
    n_chunks = n_t // T_CHUNK
    assert n_b == NUM_WORKERS * B_BLK and n_t % T_CHUNK == 0
    mesh = plsc.VectorSubcoreMesh(core_axis_name="c", subcore_axis_name="s")

    scratch = [
        pltpu.VMEM((T_CHUNK, B_BLK), jnp.int32),
        pltpu.VMEM((T_CHUNK * B_BLK, EMBED_DIM), jnp.float32),
        pltpu.VMEM((T_CHUNK, E_BLK, 8, B_BLK), jnp.float32),
        pltpu.SemaphoreType.DMA,
        pltpu.SemaphoreType.DMA,
    ]

    @functools.partial(
        pl.kernel,
        mesh=mesh,
        compiler_params=pltpu.CompilerParams(use_tc_tiling_on_sc=False,
                                             needs_layout_passes=False),
        out_type=jax.ShapeDtypeStruct((n_t, E_BLK, NUM_WORKERS, 8, B_BLK),
                                      jnp.float32),
        scratch_types=scratch,
    )
    def gather_kernel(xt_hbm, table_hbm, out_hbm, idx_v, rows_v, tiles_v,
                      gsem, wsem):
        wid = lax.axis_index("s") * NUM_CORES + lax.axis_index("c")
        b0 = wid * B_BLK

        @pl.loop(0, n_chunks)
        def _(chunk):
            t0 = chunk * T_CHUNK
            pltpu.sync_copy(xt_hbm.at[pl.ds(t0, T_CHUNK), pl.ds(b0, B_BLK)],
                            idx_v)
            for tj in range(T_CHUNK):
                pltpu.async_copy(table_hbm.at[idx_v.at[tj]],
                                 rows_v.at[pl.ds(tj * B_BLK, B_BLK)], gsem)
            for tj in range(T_CHUNK):
                pltpu.make_async_copy(
                    table_hbm.at[idx_v.at[tj]],
                    rows_v.at[pl.ds(tj * B_BLK, B_BLK)], gsem).wait()
            @pl.when(chunk > 0)
            def _():
                pltpu.make_async_copy(
                    tiles_v, out_hbm.at[pl.ds(0, T_CHUNK), :, wid],
                    wsem).wait()
            lane = lax.iota(jnp.int32, LANES)
            for tj in range(T_CHUNK):
                for g in range(B_BLK // LANES):
                    row_idx = lane + (tj * B_BLK + g * LANES)
                    for e in range(EMBED_DIM):
                        vals = plsc.load_gather(
                            rows_v, [row_idx,
                                     jnp.full((LANES,), e, jnp.int32)])
                        tiles_v[tj, e // 8, e % 8,
                                pl.ds(g * LANES, LANES)] = vals
            pltpu.async_copy(tiles_v,
                             out_hbm.at[pl.ds(t0, T_CHUNK), :, wid], wsem)

        pltpu.make_async_copy(
            tiles_v, out_hbm.at[pl.ds(0, T_CHUNK), :, wid], wsem).wait()

    return gather_kernel


def kernel(x, tool_embeddings):
    n_b, n_t = x.shape
    v5 = _make_gather(n_b, n_t)(x.T, tool_embeddings)
    return v5.transpose(2, 4, 0, 1, 3).reshape(n_b, n_t, EMBED_DIM)


# speedup vs baseline: 1.0004x; 1.0004x over previous
import functools

import jax
import jax.numpy as jnp
from jax import lax
from jax.experimental import pallas as pl
from jax.experimental.pallas import tpu as pltpu
from jax.experimental.pallas import tpu_sc as plsc

EMBED_DIM = 32
E_BLK = EMBED_DIM // 8
NUM_CORES = 2
NUM_SUBCORES = 16
NUM_WORKERS = NUM_CORES * NUM_SUBCORES
B_BLK = 128
T_CHUNK = 10
LANES = 16


@functools.lru_cache(maxsize=None)
def _make_gather(n_b, n_t):
    n_chunks = n_t // T_CHUNK
    assert n_b == NUM_WORKERS * B_BLK and n_t % T_CHUNK == 0
    mesh = plsc.VectorSubcoreMesh(core_axis_name="c", subcore_axis_name="s")

    scratch = [
        pltpu.VMEM((T_CHUNK, B_BLK), jnp.int32),
        pltpu.VMEM((T_CHUNK * B_BLK, EMBED_DIM), jnp.float32),
        pltpu.VMEM((T_CHUNK, E_BLK, 8, B_BLK), jnp.float32),
        pltpu.SemaphoreType.DMA,
        pltpu.SemaphoreType.DMA,
    ]

    @functools.partial(
        pl.kernel,
        mesh=mesh,
        compiler_params=pltpu.CompilerParams(use_tc_tiling_on_sc=False,
                                             needs_layout_passes=False,
                                             disable_bounds_checks=True),
        out_type=jax.ShapeDtypeStruct((n_t, E_BLK, NUM_WORKERS, 8, B_BLK),
                                      jnp.float32),
        scratch_types=scratch,
    )
    def gather_kernel(xt_hbm, table_hbm, out_hbm, idx_v, rows_v, tiles_v,
                      gsem, wsem):
        wid = lax.axis_index("s") * NUM_CORES + lax.axis_index("c")
        b0 = wid * B_BLK
        lane = lax.iota(jnp.int32, LANES)
        cols = [jnp.full((LANES,), e, jnp.int32) for e in range(EMBED_DIM)]

        @pl.loop(0, n_chunks)
        def _(chunk):
            t0 = chunk * T_CHUNK
            pltpu.sync_copy(xt_hbm.at[pl.ds(t0, T_CHUNK), pl.ds(b0, B_BLK)],
                            idx_v)
            for tj in range(T_CHUNK):
                pltpu.async_copy(table_hbm.at[idx_v.at[tj]],
                                 rows_v.at[pl.ds(tj * B_BLK, B_BLK)], gsem)
            for tj in range(T_CHUNK):
                pltpu.make_async_copy(
                    table_hbm.at[idx_v.at[tj]],
                    rows_v.at[pl.ds(tj * B_BLK, B_BLK)], gsem).wait()
            @pl.when(chunk > 0)
            def _():
                pltpu.make_async_copy(
                    tiles_v, out_hbm.at[pl.ds(0, T_CHUNK), :, wid],
                    wsem).wait()
            for tj in range(T_CHUNK):
                for g in range(B_BLK // LANES):
                    row_idx = lane + (tj * B_BLK + g * LANES)
                    for e in range(EMBED_DIM):
                        vals = plsc.load_gather(rows_v, [row_idx, cols[e]])
                        tiles_v[tj, e // 8, e % 8,
                                pl.ds(g * LANES, LANES)] = vals
            pltpu.async_copy(tiles_v,
                             out_hbm.at[pl.ds(t0, T_CHUNK), :, wid], wsem)

        pltpu.make_async_copy(
            tiles_v, out_hbm.at[pl.ds(0, T_CHUNK), :, wid], wsem).wait()

    return gather_kernel


def kernel(x, tool_embeddings):
    n_b, n_t = x.shape
    v5 = _make_gather(n_b, n_t)(x.T, tool_embeddings)
    return v5.transpose(2, 4, 0, 1, 3).reshape(n_b, n_t, EMBED_DIM)
